# Initial kernel scaffold; baseline (speedup 1.0000x reference)
import jax
import jax.numpy as jnp
from jax.experimental import pallas as pl


def kernel(seq1, seq2, seq3, seq4, edge_w, aug1_w, aug2_w, lbl, W0, b0, prelu0, W1, b1, prelu1, gamma0, beta0, gamma1, beta1, dgi_prompt, dgi_W, dgi_b, cl_prompt, cl_W, cl_b, lp_prompt, edge_index, aug1_edge_index, aug2_edge_index, sample):
    raise NotImplementedError("write your pallas kernel here")


# speedup vs baseline: 1.8651x; 1.8651x over previous
import jax
import jax.numpy as jnp
from jax import lax
from jax.experimental import pallas as pl
from jax.experimental.pallas import tpu as pltpu
import jax.experimental.pallas.tpu_sc as plsc

_N = 10000
_F = 128
_E = 320000
_B = 128
_NB = 80
_EP = 32 * _NB * _B
_NACC = 10240
_RPT = 640
_BLK = 1000
_GRID = _N // _BLK
_T = 100
_L = 10
_EPS = 1e-8


def _spmm_body(x_hbm, src_hbm, dst_hbm, w_hbm, out_hbm, acc, srcv, dstv, wv,
               rows, sem):
    cid = lax.axis_index("c")
    sid = lax.axis_index("s")
    tid = sid * 2 + cid

    def _zf(i, _):
        rows[i // 8, pl.ds((i % 8) * 16, 16)] = jnp.zeros((16,), jnp.float32)
        return 0
    lax.fori_loop(0, _B * 8, _zf, 0)
    for k in range(5):
        pltpu.sync_copy(rows, acc.at[pl.ds(sid * _RPT + k * _B, _B)])
    plsc.subcore_barrier()

    pltpu.sync_copy(src_hbm.at[tid], srcv)
    pltpu.sync_copy(dst_hbm.at[tid], dstv)
    pltpu.sync_copy(w_hbm.at[tid], wv)

    def _batch(j, _):
        pltpu.async_copy(x_hbm.at[srcv.at[j]], rows, sem).wait()

        def _group(g, _):
            w16 = wv[j, pl.ds(g * 16, 16)]
            for t in range(16):
                we = w16[t]
                e = g * 16 + t
                for q in range(8):
                    sl = pl.ds(q * 16, 16)
                    rows[e, sl] = rows[e, sl] * we
            return 0
        lax.fori_loop(0, _B // 16, _group, 0)
        pltpu.sync_copy(rows, acc.at[dstv.at[j]], add=True)
        return 0
    lax.fori_loop(0, _NB, _batch, 0)

    plsc.subcore_barrier()
    pltpu.sync_copy(acc.at[pl.ds(sid * _RPT, _RPT)],
                    out_hbm.at[cid, pl.ds(sid * _RPT, _RPT)])


def _spmm(x, src3, dst3, w3):
    mesh = plsc.VectorSubcoreMesh(core_axis_name="c", subcore_axis_name="s")
    f = pl.kernel(
        _spmm_body,
        out_type=jax.ShapeDtypeStruct((2, _NACC, _F), jnp.float32),
        mesh=mesh,
        scratch_types=[
            pltpu.VMEM_SHARED((_NACC, _F), jnp.float32),
            pltpu.VMEM((_NB, _B), jnp.int32),
            pltpu.VMEM((_NB, _B), jnp.int32),
            pltpu.VMEM((_NB, _B), jnp.float32),
            pltpu.VMEM((_B, _F), jnp.float32),
            pltpu.SemaphoreType.DMA,
        ],
    )
    return f(x, src3, dst3, w3)


def _gather_body(x_hbm, idx_hbm, out_hbm, idxv, rows, sem):
    cid = lax.axis_index("c")
    sid = lax.axis_index("s")
    tid = sid * 2 + cid

    @pl.when(tid < 25)
    def _():
        pltpu.sync_copy(idx_hbm.at[tid], idxv)
        pltpu.async_copy(x_hbm.at[idxv], rows, sem).wait()
        pltpu.sync_copy(rows, out_hbm.at[pl.ds(tid * 80, 80)])


def _gather_rows(x, idx2):
    mesh = plsc.VectorSubcoreMesh(core_axis_name="c", subcore_axis_name="s")
    f = pl.kernel(
        _gather_body,
        out_type=jax.ShapeDtypeStruct((2 * _T * _L, _F), jnp.float32),
        mesh=mesh,
        scratch_types=[
            pltpu.VMEM((80,), jnp.int32),
            pltpu.VMEM((80, _F), jnp.float32),
            pltpu.SemaphoreType.DMA,
        ],
    )
    return f(x, idx2)


def _prep_edges(ei, w):
    pad = _EP - _E
    src = jnp.concatenate([ei[0], jnp.zeros((pad,), jnp.int32)])
    dst = jnp.concatenate([ei[1], jnp.zeros((pad,), jnp.int32)])
    wp = jnp.concatenate([w, jnp.zeros((pad,), jnp.float32)])
    return (src.reshape(32, _NB, _B), dst.reshape(32, _NB, _B),
            wp.reshape(32, _NB, _B))


def _mm_pre_body(s1_ref, s2_ref, w0_ref, o1_ref, o2_ref):
    w0 = w0_ref[...]
    o1_ref[...] = jnp.dot(s1_ref[...], w0, preferred_element_type=jnp.float32)
    o2_ref[...] = jnp.dot(s2_ref[...], w0, preferred_element_type=jnp.float32)


def _mm_pre(s1, s2, w0):
    blk = pl.BlockSpec((_BLK, _F), lambda i: (i, 0))
    return pl.pallas_call(
        _mm_pre_body,
        grid=(_GRID,),
        in_specs=[blk, blk, pl.BlockSpec((_F, _F), lambda i: (0, 0))],
        out_specs=[blk, blk],
        out_shape=[jax.ShapeDtypeStruct((_N, _F), jnp.float32)] * 2,
    )(s1, s2, w0)


def _act1_body(ye1, ye2, ya, yb, b0, a0, z1o, h2o, hao, hbo, so):
    b = b0[...]
    a = a0[0, 0]

    def act(ref):
        h = ref[0] + ref[1] + b
        return jnp.where(h > 0, h, a * h)

    z1 = act(ye1)
    z1o[...] = z1
    h2o[...] = act(ye2)
    hao[...] = act(ya)
    hbo[...] = act(yb)
    zsum = jnp.sum(z1, axis=0, keepdims=True)
    zsq = jnp.sum(z1 * z1, axis=0, keepdims=True)
    pad = jnp.zeros((6, _F), jnp.float32)
    s = jnp.concatenate([zsum, zsq, pad], axis=0)

    @pl.when(pl.program_id(0) == 0)
    def _():
        so[...] = s

    @pl.when(pl.program_id(0) != 0)
    def _():
        so[...] = so[...] + s


def _act1(ye1, ye2, ya, yb, b0, a0):
    pblk = pl.BlockSpec((2, _BLK, _F), lambda i: (0, i, 0))
    blk = pl.BlockSpec((_BLK, _F), lambda i: (i, 0))
    one = pl.BlockSpec((1, _F), lambda i: (0, 0))
    sc = pl.BlockSpec((1, 1), lambda i: (0, 0))
    return pl.pallas_call(
        _act1_body,
        grid=(_GRID,),
        in_specs=[pblk, pblk, pblk, pblk, one, sc],
        out_specs=[blk, blk, blk, blk, pl.BlockSpec((8, _F), lambda i: (0, 0))],
        out_shape=[jax.ShapeDtypeStruct((_N, _F), jnp.float32)] * 4
        + [jax.ShapeDtypeStruct((8, _F), jnp.float32)],
    )(ye1, ye2, ya, yb, b0, a0)


def _mm2_body(z1, h21, ha1, hb1, sums, g0, be0, w1_ref, oe1, oe2, ol, oa, ob):
    w1 = w1_ref[...]
    mu = sums[0:1, :] * (1.0 / _N)
    var = sums[1:2, :] * (1.0 / _N) - mu * mu
    inv = lax.rsqrt(var + 1e-5)
    hl1 = (z1[...] - mu) * inv * g0[...] + be0[...]

    def mm(x):
        return jnp.dot(x, w1, preferred_element_type=jnp.float32)

    oe1[...] = mm(z1[...])
    oe2[...] = mm(h21[...])
    ol[...] = mm(hl1)
    oa[...] = mm(ha1[...])
    ob[...] = mm(hb1[...])


def _mm2(z1, h21, ha1, hb1, sums, g0, be0, w1):
    blk = pl.BlockSpec((_BLK, _F), lambda i: (i, 0))
    one = pl.BlockSpec((1, _F), lambda i: (0, 0))
    return pl.pallas_call(
        _mm2_body,
        grid=(_GRID,),
        in_specs=[blk, blk, blk, blk,
                  pl.BlockSpec((8, _F), lambda i: (0, 0)), one, one,
                  pl.BlockSpec((_F, _F), lambda i: (0, 0))],
        out_specs=[blk] * 5,
        out_shape=[jax.ShapeDtypeStruct((_N, _F), jnp.float32)] * 5,
    )(z1, h21, ha1, hb1, sums, g0, be0, w1)


def _act2_body(ze1, ze2, zl, za, zb, b1, a1, h1o, h2o, hlo, so):
    b = b1[...]
    a = a1[0, 0]

    def act(ref):
        h = ref[0] + ref[1] + b
        return jnp.where(h > 0, h, a * h)

    h1 = act(ze1)
    h2 = act(ze2)
    hlp = act(zl)
    ha = act(za)
    hb = act(zb)
    h1o[...] = h1
    h2o[...] = h2
    hlo[...] = hlp
    s = jnp.concatenate([
        jnp.sum(h1, axis=0, keepdims=True),
        jnp.sum(ha, axis=0, keepdims=True),
        jnp.sum(hb, axis=0, keepdims=True),
        jnp.sum(hlp, axis=0, keepdims=True),
        jnp.sum(hlp * hlp, axis=0, keepdims=True),
        jnp.zeros((3, _F), jnp.float32),
    ], axis=0)

    @pl.when(pl.program_id(0) == 0)
    def _():
        so[...] = s

    @pl.when(pl.program_id(0) != 0)
    def _():
        so[...] = so[...] + s


def _act2(ze1, ze2, zl, za, zb, b1, a1):
    pblk = pl.BlockSpec((2, _BLK, _F), lambda i: (0, i, 0))
    blk = pl.BlockSpec((_BLK, _F), lambda i: (i, 0))
    one = pl.BlockSpec((1, _F), lambda i: (0, 0))
    sc = pl.BlockSpec((1, 1), lambda i: (0, 0))
    return pl.pallas_call(
        _act2_body,
        grid=(_GRID,),
        in_specs=[pblk, pblk, pblk, pblk, pblk, one, sc],
        out_specs=[blk, blk, blk, pl.BlockSpec((8, _F), lambda i: (0, 0))],
        out_shape=[jax.ShapeDtypeStruct((_N, _F), jnp.float32)] * 3
        + [jax.ShapeDtypeStruct((8, _F), jnp.float32)],
    )(ze1, ze2, zl, za, zb, b1, a1)


def _softplus(x):
    return jnp.maximum(x, 0.0) + jnp.log(1.0 + jnp.exp(-jnp.abs(x)))


def _heads_body(h1r, h2r, hlr, sums, dgiW, clW, dgip, clp, lpp, dgib, clb,
                g1, be1, l3o, lso):
    h1 = h1r[...]
    h2 = h2r[...]
    hlp = hlr[...]
    c = jax.nn.sigmoid(sums[0:1, :] * (1.0 / _N))
    c1 = jax.nn.sigmoid(sums[1:2, :] * (1.0 / _N) * clp[...])
    c3 = jax.nn.sigmoid(sums[2:3, :] * (1.0 / _N) * clp[...])
    dn = (((1,), (1,)), ((), ()))
    u_dgi = lax.dot_general(c, dgiW[...], dn,
                            preferred_element_type=jnp.float32)
    u_cl = lax.dot_general(c1 + c3, clW[...], dn,
                           preferred_element_type=jnp.float32)

    s1 = jnp.sum(h1 * dgip[...] * u_dgi, axis=1, keepdims=True) + dgib[0, 0]
    s2 = jnp.sum(h2 * dgip[...] * u_dgi, axis=1, keepdims=True) + dgib[0, 0]
    t1 = jnp.sum(h1 * clp[...] * u_cl, axis=1, keepdims=True) + 2.0 * clb[0, 0]
    t2 = jnp.sum(h2 * clp[...] * u_cl, axis=1, keepdims=True) + 2.0 * clb[0, 0]
    dgi_part = jnp.sum(_softplus(-s1)) + jnp.sum(_softplus(s2))
    cl_part = jnp.sum(_softplus(-t1)) + jnp.sum(_softplus(t2))

    mu = sums[3:4, :] * (1.0 / _N)
    var = sums[4:5, :] * (1.0 / _N) - mu * mu
    hl = (hlp - mu) * lax.rsqrt(var + 1e-5) * g1[...] + be1[...]
    v = hl * lpp[...]
    l3o[...] = jnp.where(v > 0, v, jnp.exp(v) - 1.0)

    lane = lax.broadcasted_iota(jnp.int32, (1, _F), 1)
    contrib = (jnp.where(lane == 0, dgi_part, 0.0)
               + jnp.where(lane == 1, cl_part, 0.0))

    @pl.when(pl.program_id(0) == 0)
    def _():
        lso[...] = contrib

    @pl.when(pl.program_id(0) != 0)
    def _():
        lso[...] = lso[...] + contrib


def _heads(h1, h2, hlp, sums, dgiW, clW, dgip, clp, lpp, dgib, clb, g1, be1):
    blk = pl.BlockSpec((_BLK, _F), lambda i: (i, 0))
    one = pl.BlockSpec((1, _F), lambda i: (0, 0))
    sc = pl.BlockSpec((1, 1), lambda i: (0, 0))
    full = pl.BlockSpec((_F, _F), lambda i: (0, 0))
    return pl.pallas_call(
        _heads_body,
        grid=(_GRID,),
        in_specs=[blk, blk, blk, pl.BlockSpec((8, _F), lambda i: (0, 0)),
                  full, full, one, one, one, sc, sc, one, one],
        out_specs=[blk, one],
        out_shape=[jax.ShapeDtypeStruct((_N, _F), jnp.float32),
                   jax.ShapeDtypeStruct((1, _F), jnp.float32)],
    )(h1, h2, hlp, sums, dgiW, clW, dgip, clp, lpp, dgib, clb, g1, be1)


def _lp_final_body(ht_ref, hi_ref, ls_ref, out_ref):
    ht = ht_ref[...]
    hi = hi_ref[...]
    num = jnp.sum(hi * ht, axis=1, keepdims=True)
    na = jnp.sqrt(jnp.sum(hi * hi, axis=1, keepdims=True))
    nb = jnp.sqrt(jnp.sum(ht * ht, axis=1, keepdims=True))
    sim = num / (jnp.maximum(na, _EPS) * jnp.maximum(nb, _EPS))
    ex = jnp.exp(sim) * (1.0 / 1.5)
    r = lax.broadcasted_iota(jnp.int32, (_T * _L, 1), 0)
    mask0 = (r % _L) == 0
    part1 = -jnp.sum(jnp.where(mask0, jnp.log(ex), 0.0))
    ri = lax.broadcasted_iota(jnp.int32, (_T, _T * _L), 0)
    rc = lax.broadcasted_iota(jnp.int32, (_T, _T * _L), 1)
    sel = ((rc >= ri * _L) & (rc < (ri + 1) * _L)
           & ((rc % _L) != 0)).astype(jnp.float32)
    den = jnp.dot(sel, ex, preferred_element_type=jnp.float32)
    part2 = jnp.sum(jnp.log(den))
    lp = (part1 + part2) * (1.0 / _T)
    dgi = ls_ref[0, 0] * (1.0 / (2 * _N))
    cl = ls_ref[0, 1] * (1.0 / (2 * _N))
    total = 0.5 * dgi + 0.3 * cl + 0.2 * lp
    out_ref[...] = jnp.broadcast_to(total, (1, 1))


def _lp_final(rows_t, rows_i, lsum):
    return pl.pallas_call(
        _lp_final_body,
        out_shape=jax.ShapeDtypeStruct((1, 1), jnp.float32),
    )(rows_t, rows_i, lsum)


def kernel(seq1, seq2, seq3, seq4, edge_w, aug1_w, aug2_w, lbl, W0, b0,
           prelu0, W1, b1, prelu1, gamma0, beta0, gamma1, beta1, dgi_prompt,
           dgi_W, dgi_b, cl_prompt, cl_W, cl_b, lp_prompt, edge_index,
           aug1_edge_index, aug2_edge_index, sample):
    s1 = seq1[0]
    s2 = seq2[0]
    b0r = b0.reshape(1, _F)
    b1r = b1.reshape(1, _F)
    a0 = prelu0.reshape(1, 1)
    a1 = prelu1.reshape(1, 1)
    g0 = gamma0.reshape(1, _F)
    be0 = beta0.reshape(1, _F)
    g1 = gamma1.reshape(1, _F)
    be1 = beta1.reshape(1, _F)
    dgib = dgi_b.reshape(1, 1)
    clb = cl_b.reshape(1, 1)

    e_e = _prep_edges(edge_index, edge_w)
    e_a = _prep_edges(aug1_edge_index, aug1_w)
    e_b = _prep_edges(aug2_edge_index, aug2_w)

    xw1, xw2 = _mm_pre(s1, s2, W0)

    ye1 = _spmm(xw1, *e_e)
    ye2 = _spmm(xw2, *e_e)
    ya = _spmm(xw1, *e_a)
    yb = _spmm(xw1, *e_b)

    z1, h21, ha1, hb1, bnsums = _act1(ye1, ye2, ya, yb, b0r, a0)
    xe1, xe2, xl, xa, xb = _mm2(z1, h21, ha1, hb1, bnsums, g0, be0, W1)

    ze1 = _spmm(xe1, *e_e)
    ze2 = _spmm(xe2, *e_e)
    zl = _spmm(xl, *e_e)
    za = _spmm(xa, *e_a)
    zb = _spmm(xb, *e_b)

    h1, h2, hlp, sums5 = _act2(ze1, ze2, zl, za, zb, b1r, a1)
    logits3, lsum = _heads(h1, h2, hlp, sums5, dgi_W, cl_W, dgi_prompt,
                           cl_prompt, lp_prompt, dgib, clb, g1, be1)

    idx_t = sample.reshape(-1)
    idx_i = jnp.repeat(jnp.arange(_T, dtype=jnp.int32), _L)
    idx_all = jnp.concatenate([idx_t, idx_i]).reshape(25, 80)
    rows_all = _gather_rows(logits3, idx_all)
    rows_t = rows_all[: _T * _L]
    rows_i = rows_all[_T * _L:]

    out = _lp_final(rows_t, rows_i, lsum)
    return out.reshape(())
